# Initial kernel scaffold; baseline (speedup 1.0000x reference)
import jax
import jax.numpy as jnp
from jax.experimental import pallas as pl


def kernel(x_user, x_movie, edge_src, edge_dst, Wu, bu, Wm, bm, l0_u2m_Wl, l0_u2m_bl, l0_u2m_Wr, l0_m2u_Wl, l0_m2u_bl, l0_m2u_Wr, l1_u2m_Wl, l1_u2m_bl, l1_u2m_Wr, l1_m2u_Wl, l1_m2u_bl, l1_m2u_Wr):
    raise NotImplementedError("write your pallas kernel here")


# speedup vs baseline: 5.0931x; 5.0931x over previous
import functools

import jax
import jax.numpy as jnp
from jax import lax
from jax.experimental import pallas as pl
from jax.experimental.pallas import tpu as pltpu
from jax.experimental.pallas import tpu_sc as plsc

NU, NM, NE, DIN, DH = 50000, 10000, 800000, 128, 64
NC, NS = 2, 16
NW = NC * NS
CHUNK = 128
NCHUNK = NE // CHUNK
UHALF = NU // NC
UPAD = 25088
MPAD = 10240
DW = 8

_mesh = lambda: plsc.VectorSubcoreMesh(core_axis_name="c", subcore_axis_name="s")


def _sc_movie_pass(with_deg):
    out_type = [jax.ShapeDtypeStruct((NC, MPAD, DH), jnp.float32)]
    scratch = [
        pltpu.VMEM((CHUNK,), jnp.int32),
        pltpu.VMEM((CHUNK,), jnp.int32),
        pltpu.VMEM((CHUNK, DH), jnp.float32),
        pltpu.VMEM((CHUNK, DW), jnp.float32),
        pltpu.VMEM_SHARED((MPAD, DH), jnp.float32),
        pltpu.VMEM_SHARED((MPAD, DW), jnp.float32),
        pltpu.SemaphoreType.DMA,
    ]
    if with_deg:
        out_type.append(jax.ShapeDtypeStruct((NC, MPAD, DW), jnp.float32))

    def body(*refs):
        if with_deg:
            (x_hbm, src_hbm, dst_hbm, z64_hbm, z8_hbm, ones_hbm,
             out_hbm, deg_hbm,
             idx_v, dst_v, rows_v, ones_v, acc, dacc, sem) = refs
        else:
            (x_hbm, src_hbm, dst_hbm, z64_hbm, z8_hbm, ones_hbm,
             out_hbm,
             idx_v, dst_v, rows_v, ones_v, acc, dacc, sem) = refs
        c = lax.axis_index("c")
        s = lax.axis_index("s")
        w = s * NC + c
        zr = MPAD // NS
        pltpu.sync_copy(z64_hbm.at[pl.ds(s * zr, zr)], acc.at[pl.ds(s * zr, zr)])
        if with_deg:
            pltpu.sync_copy(z8_hbm.at[pl.ds(s * zr, zr)], dacc.at[pl.ds(s * zr, zr)])
            pltpu.sync_copy(ones_hbm, ones_v)
        plsc.subcore_barrier()

        nt = (NCHUNK + NW - 1) // NW

        def step(t, carry):
            g = w + NW * t

            @pl.when(g < NCHUNK)
            def _():
                base = g * CHUNK
                pltpu.sync_copy(src_hbm.at[pl.ds(base, CHUNK)], idx_v)
                pltpu.async_copy(x_hbm.at[idx_v], rows_v, sem).wait()
                pltpu.sync_copy(dst_hbm.at[pl.ds(base, CHUNK)], dst_v)
                pltpu.sync_copy(rows_v, acc.at[dst_v], add=True)
                if with_deg:
                    pltpu.sync_copy(ones_v, dacc.at[dst_v], add=True)

            return carry

        lax.fori_loop(0, nt, step, 0)
        plsc.subcore_barrier()
        pltpu.sync_copy(acc.at[pl.ds(s * zr, zr)], out_hbm.at[c, pl.ds(s * zr, zr)])
        if with_deg:
            pltpu.sync_copy(dacc.at[pl.ds(s * zr, zr)], deg_hbm.at[c, pl.ds(s * zr, zr)])

    return pl.kernel(body, out_type=tuple(out_type), mesh=_mesh(),
                     scratch_types=tuple(scratch),
                     compiler_params=pltpu.CompilerParams(use_tc_tiling_on_sc=False))


def _sc_user_pass(with_deg):
    out_type = [jax.ShapeDtypeStruct((NC, UPAD, DH), jnp.float32)]
    scratch = [
        pltpu.VMEM((CHUNK,), jnp.int32),
        pltpu.VMEM((CHUNK,), jnp.int32),
        pltpu.VMEM((CHUNK, DH), jnp.float32),
        pltpu.VMEM((CHUNK, DW), jnp.float32),
        pltpu.VMEM_SHARED((UPAD, DH), jnp.float32),
        pltpu.VMEM_SHARED((UPAD, DW), jnp.float32),
        pltpu.SemaphoreType.DMA,
    ]
    if with_deg:
        out_type.append(jax.ShapeDtypeStruct((NC, UPAD, DW), jnp.float32))

    def body(*refs):
        if with_deg:
            (x_hbm, src_hbm, dst_hbm, z64_hbm, z8_hbm, ones_hbm,
             out_hbm, deg_hbm,
             idx_v, dst_v, rows_v, ones_v, acc, dacc, sem) = refs
        else:
            (x_hbm, src_hbm, dst_hbm, z64_hbm, z8_hbm, ones_hbm,
             out_hbm,
             idx_v, dst_v, rows_v, ones_v, acc, dacc, sem) = refs
        c = lax.axis_index("c")
        s = lax.axis_index("s")
        zr = UPAD // NS
        pltpu.sync_copy(z64_hbm.at[pl.ds(s * zr, zr)], acc.at[pl.ds(s * zr, zr)])
        if with_deg:
            pltpu.sync_copy(z8_hbm.at[pl.ds(s * zr, zr)], dacc.at[pl.ds(s * zr, zr)])
            pltpu.sync_copy(ones_hbm, ones_v)
        plsc.subcore_barrier()

        lo = c * UHALF
        nt = (NCHUNK + NS - 1) // NS

        def step(t, carry):
            g = s + NS * t

            @pl.when(g < NCHUNK)
            def _():
                base = g * CHUNK
                pltpu.sync_copy(src_hbm.at[pl.ds(base, CHUNK)], idx_v)
                pltpu.async_copy(x_hbm.at[idx_v], rows_v, sem).wait()
                pltpu.sync_copy(dst_hbm.at[pl.ds(base, CHUNK)], dst_v)
                for i in range(CHUNK // 16):
                    v = dst_v[pl.ds(i * 16, 16)] - lo
                    ok = (v >= 0) & (v < UHALF)
                    dst_v[pl.ds(i * 16, 16)] = jnp.where(ok, v, UHALF)
                pltpu.sync_copy(rows_v, acc.at[dst_v], add=True)
                if with_deg:
                    pltpu.sync_copy(ones_v, dacc.at[dst_v], add=True)

            return carry

        lax.fori_loop(0, nt, step, 0)
        plsc.subcore_barrier()
        pltpu.sync_copy(acc.at[pl.ds(s * zr, zr)], out_hbm.at[c, pl.ds(s * zr, zr)])
        if with_deg:
            pltpu.sync_copy(dacc.at[pl.ds(s * zr, zr)], deg_hbm.at[c, pl.ds(s * zr, zr)])

    return pl.kernel(body, out_type=tuple(out_type), mesh=_mesh(),
                     scratch_types=tuple(scratch),
                     compiler_params=pltpu.CompilerParams(use_tc_tiling_on_sc=False))


_movie_pass_deg = _sc_movie_pass(True)
_movie_pass = _sc_movie_pass(False)
_user_pass_deg = _sc_user_pass(True)
_user_pass = _sc_user_pass(False)

_BLK = 400


def _tc_proj(x, w, b):
    n, d = x.shape

    def body(x_ref, w_ref, b_ref, o_ref):
        o_ref[...] = jnp.dot(x_ref[...], w_ref[...],
                             preferred_element_type=jnp.float32) + b_ref[...]

    return pl.pallas_call(
        body,
        grid=(n // _BLK,),
        in_specs=[pl.BlockSpec((_BLK, d), lambda i: (i, 0)),
                  pl.BlockSpec((d, DH), lambda i: (0, 0)),
                  pl.BlockSpec((1, DH), lambda i: (0, 0))],
        out_specs=pl.BlockSpec((_BLK, DH), lambda i: (i, 0)),
        out_shape=jax.ShapeDtypeStruct((n, DH), jnp.float32),
    )(x, w, b.reshape(1, DH))


def _tc_combine2(s0, s1, d0, d1, xd, wl, bl, wr):
    n = xd.shape[0]

    def body(s0_r, s1_r, d0_r, d1_r, xd_r, wl_r, bl_r, wr_r, o_ref):
        deg = d0_r[...][:, :1] + d1_r[...][:, :1]
        mean = (s0_r[...] + s1_r[...]) / jnp.maximum(deg, 1.0)
        acc = jnp.dot(mean, wl_r[...], preferred_element_type=jnp.float32)
        acc += jnp.dot(xd_r[...], wr_r[...], preferred_element_type=jnp.float32)
        o_ref[...] = jnp.maximum(acc + bl_r[...], 0.0)

    return pl.pallas_call(
        body,
        grid=(n // _BLK,),
        in_specs=[pl.BlockSpec((_BLK, DH), lambda i: (i, 0)),
                  pl.BlockSpec((_BLK, DH), lambda i: (i, 0)),
                  pl.BlockSpec((_BLK, DW), lambda i: (i, 0)),
                  pl.BlockSpec((_BLK, DW), lambda i: (i, 0)),
                  pl.BlockSpec((_BLK, DH), lambda i: (i, 0)),
                  pl.BlockSpec((DH, DH), lambda i: (0, 0)),
                  pl.BlockSpec((1, DH), lambda i: (0, 0)),
                  pl.BlockSpec((DH, DH), lambda i: (0, 0))],
        out_specs=pl.BlockSpec((_BLK, DH), lambda i: (i, 0)),
        out_shape=jax.ShapeDtypeStruct((n, DH), jnp.float32),
    )(s0, s1, d0, d1, xd, wl, bl.reshape(1, DH), wr)


def _tc_combine1(su, du, xd, wl, bl, wr):
    n = xd.shape[0]

    def body(su_r, du_r, xd_r, wl_r, bl_r, wr_r, o_ref):
        deg = du_r[...][:, :1]
        mean = su_r[...] / jnp.maximum(deg, 1.0)
        acc = jnp.dot(mean, wl_r[...], preferred_element_type=jnp.float32)
        acc += jnp.dot(xd_r[...], wr_r[...], preferred_element_type=jnp.float32)
        o_ref[...] = jnp.maximum(acc + bl_r[...], 0.0)

    return pl.pallas_call(
        body,
        grid=(n // _BLK,),
        in_specs=[pl.BlockSpec((_BLK, DH), lambda i: (i, 0)),
                  pl.BlockSpec((_BLK, DW), lambda i: (i, 0)),
                  pl.BlockSpec((_BLK, DH), lambda i: (i, 0)),
                  pl.BlockSpec((DH, DH), lambda i: (0, 0)),
                  pl.BlockSpec((1, DH), lambda i: (0, 0)),
                  pl.BlockSpec((DH, DH), lambda i: (0, 0))],
        out_specs=pl.BlockSpec((_BLK, DH), lambda i: (i, 0)),
        out_shape=jax.ShapeDtypeStruct((n, DH), jnp.float32),
    )(su, du, xd, wl, bl.reshape(1, DH), wr)


def kernel(x_user, x_movie, edge_src, edge_dst, Wu, bu, Wm, bm,
           l0_u2m_Wl, l0_u2m_bl, l0_u2m_Wr, l0_m2u_Wl, l0_m2u_bl, l0_m2u_Wr,
           l1_u2m_Wl, l1_u2m_bl, l1_u2m_Wr, l1_m2u_Wl, l1_m2u_bl, l1_m2u_Wr):
    z64 = jnp.zeros((UPAD, DH), jnp.float32)
    z8 = jnp.zeros((UPAD, DW), jnp.float32)
    ones = jnp.ones((CHUNK, DW), jnp.float32)

    xu = _tc_proj(x_user, Wu, bu)
    xm = _tc_proj(x_movie, Wm, bm)

    msum, mdeg = _movie_pass_deg(xu, edge_src, edge_dst, z64, z8, ones)
    usum, udeg = _user_pass_deg(xm, edge_dst, edge_src, z64, z8, ones)
    usum_full = jnp.concatenate([usum[0, :UHALF], usum[1, :UHALF]], axis=0)
    udeg_full = jnp.concatenate([udeg[0, :UHALF], udeg[1, :UHALF]], axis=0)
    new_m = _tc_combine2(msum[0, :NM], msum[1, :NM], mdeg[0, :NM], mdeg[1, :NM], xm,
                         l0_u2m_Wl, l0_u2m_bl, l0_u2m_Wr)
    new_u = _tc_combine1(usum_full, udeg_full, xu,
                         l0_m2u_Wl, l0_m2u_bl, l0_m2u_Wr)
    xu, xm = new_u, new_m

    (msum,) = _movie_pass(xu, edge_src, edge_dst, z64, z8, ones)
    (usum,) = _user_pass(xm, edge_dst, edge_src, z64, z8, ones)
    usum_full = jnp.concatenate([usum[0, :UHALF], usum[1, :UHALF]], axis=0)
    new_m = _tc_combine2(msum[0, :NM], msum[1, :NM], mdeg[0, :NM], mdeg[1, :NM], xm,
                         l1_u2m_Wl, l1_u2m_bl, l1_u2m_Wr)
    new_u = _tc_combine1(usum_full, udeg_full, xu,
                         l1_m2u_Wl, l1_m2u_bl, l1_m2u_Wr)
    return (new_u, new_m)
